# baseline (speedup 1.0000x reference)
import functools

import jax
import jax.numpy as jnp
from jax import lax
from jax.experimental import pallas as pl
from jax.experimental.pallas import tpu as pltpu
from jax.experimental.pallas import tpu_sc as plsc

NUM_CORES = 2
NUM_SUBCORES = 16
LANES = 16
NW = NUM_CORES * NUM_SUBCORES

NUM_USERS_PAD = 1000064
NUM_ITEMS_PAD = 100096

BATCH = 16384
EMBED = 64
B_PER_W = BATCH // NW
CHUNK = 128
N_CHUNKS = B_PER_W // CHUNK
HALF = B_PER_W // 2
N_GROUPS = HALF // LANES
GPC = CHUNK // LANES


def _mf_body(u_idx_hbm, i_idx_hbm, u_tab, ub_tab, i_tab, ib_tab, mean_hbm,
             out_hbm,
             u_idx_v, i_idx_v, u_row_v, i_row_v, ub_row_v, ib_row_v,
             u_col_v, i_col_v, ub_col_v, ib_col_v,
             u_buf, i_buf, bu_v, bi_v, mean_v, out_v, out_t, sem, bsem):
    cid = lax.axis_index("c")
    sid = lax.axis_index("s")
    wid = sid * NUM_CORES + cid

    pltpu.sync_copy(u_idx_hbm.at[wid], u_idx_v)
    pltpu.sync_copy(i_idx_hbm.at[wid], i_idx_v)
    pltpu.sync_copy(mean_hbm, mean_v)

    for j in range(N_CHUNKS):
        for g in range(GPC):
            sl = pl.ds(g * LANES, LANES)
            fsl = pl.ds(j * CHUNK + g * LANES, LANES)
            u = u_idx_v[j, sl]
            i = i_idx_v[j, sl]
            u_row_v[j, sl] = lax.shift_right_logical(u, 1)
            i_row_v[j, sl] = lax.shift_right_logical(i, 1)
            ub_row_v[j, sl] = lax.shift_right_logical(u, 7)
            ib_row_v[j, sl] = lax.shift_right_logical(i, 7)
            u_col_v[fsl] = lax.shift_left(jnp.bitwise_and(u, 1), 6)
            i_col_v[fsl] = lax.shift_left(jnp.bitwise_and(i, 1), 6)
            ub_col_v[fsl] = jnp.bitwise_and(u, CHUNK - 1)
            ib_col_v[fsl] = jnp.bitwise_and(i, CHUNK - 1)

    mean_vec = mean_v[...]

    for p in range(2):
        copies = []
        for jj in range(2):
            j = p * 2 + jj
            dst = pl.ds(jj * CHUNK, CHUNK)
            copies.append(
                pltpu.async_copy(u_tab.at[u_row_v.at[j]], u_buf.at[dst], sem))
            copies.append(
                pltpu.async_copy(i_tab.at[i_row_v.at[j]], i_buf.at[dst], sem))
        for c in copies:
            c.wait()

        def group(g, carry, p=p):
            row = g * LANES + lax.iota(jnp.int32, LANES)
            fsl = pl.ds(p * HALF + g * LANES, LANES)
            ucol = u_col_v[fsl]
            icol = i_col_v[fsl]
            acc = mean_vec
            for d in range(EMBED):
                acc = acc + (plsc.load_gather(u_buf, [row, ucol + d])
                             * plsc.load_gather(i_buf, [row, icol + d]))
            out_v[fsl] = acc
            return carry

        lax.fori_loop(0, N_GROUPS, group, 0)

    for row_ref, col_ref, dst_ref in ((ub_row_v, ub_col_v, bu_v),
                                      (ib_row_v, ib_col_v, bi_v)):
        tab = ub_tab if dst_ref is bu_v else ib_tab
        copies = []
        for j in range(N_CHUNKS):
            buf = u_buf if j < 2 else i_buf
            dst = pl.ds((j % 2) * CHUNK, CHUNK)
            copies.append(
                pltpu.async_copy(tab.at[row_ref.at[j]], buf.at[dst], bsem))
        for c in copies:
            c.wait()
        for g in range(B_PER_W // LANES):
            buf = u_buf if g < 2 * GPC else i_buf
            row = (g % (2 * GPC)) * LANES + lax.iota(jnp.int32, LANES)
            fsl = pl.ds(g * LANES, LANES)
            dst_ref[fsl] = plsc.load_gather(buf, [row, col_ref[fsl]])

    for g in range(B_PER_W // LANES):
        sl = pl.ds(g * LANES, LANES)
        out_t[g // GPC, pl.ds((g % GPC) * LANES, LANES)] = (
            out_v[sl] + bu_v[sl] + bi_v[sl])

    pltpu.sync_copy(out_t, out_hbm.at[pl.ds(wid * N_CHUNKS, N_CHUNKS)])


@jax.jit
def _mf_sc(u_idx, i_idx, u_tab, ub_tab, i_tab, ib_tab, mean):
    mesh = plsc.VectorSubcoreMesh(core_axis_name="c", subcore_axis_name="s",
                                  num_cores=NUM_CORES, num_subcores=NUM_SUBCORES)
    run = functools.partial(
        pl.kernel,
        out_type=jax.ShapeDtypeStruct((NW * N_CHUNKS, CHUNK), jnp.float32),
        mesh=mesh,
        scratch_types=[
            pltpu.VMEM((N_CHUNKS, CHUNK), jnp.int32),
            pltpu.VMEM((N_CHUNKS, CHUNK), jnp.int32),
            pltpu.VMEM((N_CHUNKS, CHUNK), jnp.int32),
            pltpu.VMEM((N_CHUNKS, CHUNK), jnp.int32),
            pltpu.VMEM((N_CHUNKS, CHUNK), jnp.int32),
            pltpu.VMEM((N_CHUNKS, CHUNK), jnp.int32),
            pltpu.VMEM((B_PER_W,), jnp.int32),
            pltpu.VMEM((B_PER_W,), jnp.int32),
            pltpu.VMEM((B_PER_W,), jnp.int32),
            pltpu.VMEM((B_PER_W,), jnp.int32),
            pltpu.VMEM((HALF, CHUNK), jnp.float32),
            pltpu.VMEM((HALF, CHUNK), jnp.float32),
            pltpu.VMEM((B_PER_W,), jnp.float32),
            pltpu.VMEM((B_PER_W,), jnp.float32),
            pltpu.VMEM((LANES,), jnp.float32),
            pltpu.VMEM((B_PER_W,), jnp.float32),
            pltpu.VMEM((N_CHUNKS, CHUNK), jnp.float32),
            pltpu.SemaphoreType.DMA,
            pltpu.SemaphoreType.DMA,
        ],
        compiler_params=pltpu.CompilerParams(needs_layout_passes=False,
                                             use_tc_tiling_on_sc=True),
    )(_mf_body)
    return run(u_idx, i_idx, u_tab, ub_tab, i_tab, ib_tab, mean)


def kernel(u_id, i_id, user_emb, user_bias, item_emb, item_bias, mean):
    u_idx = u_id.astype(jnp.int32).reshape(NW, N_CHUNKS, CHUNK)
    i_idx = i_id.astype(jnp.int32).reshape(NW, N_CHUNKS, CHUNK)
    u_tab = user_emb.reshape(-1, 2 * EMBED)
    i_tab = item_emb.reshape(-1, 2 * EMBED)
    ub_tab = jnp.pad(user_bias.reshape(-1),
                     (0, NUM_USERS_PAD - user_bias.shape[0])).reshape(-1, CHUNK)
    ib_tab = jnp.pad(item_bias.reshape(-1),
                     (0, NUM_ITEMS_PAD - item_bias.shape[0])).reshape(-1, CHUNK)
    mean16 = jnp.broadcast_to(mean.astype(jnp.float32).reshape(1), (LANES,))
    out = _mf_sc(u_idx, i_idx, u_tab, ub_tab, i_tab, ib_tab, mean16)
    return out.reshape(BATCH)


# speedup vs baseline: 1.8363x; 1.8363x over previous
import functools

import jax
import jax.numpy as jnp
from jax import lax
from jax.experimental import pallas as pl
from jax.experimental.pallas import tpu as pltpu
from jax.experimental.pallas import tpu_sc as plsc

NUM_CORES = 2
NUM_SUBCORES = 16
LANES = 16

NUM_USERS = 1000000
NUM_ITEMS = 100000
NUM_USERS_PAD = 1000064
NUM_ITEMS_PAD = 100096
BATCH = 16384
EMBED = 64

B_PER_S = BATCH // NUM_SUBCORES
CHUNK = 128
N_CHUNKS = B_PER_S // CHUNK
D_PER_C = EMBED // NUM_CORES
GROUPS = B_PER_S // LANES


def _mf_body(uT, i_tab, ub_tab, ib_tab, u_idx_hbm, i_idx_hbm, i_row_hbm,
             i_col_hbm, mean_hbm, out_hbm,
             u_idx_v, i_idx_v, i_row_v, i_col_v, ubrow_v, ibrow_v,
             ubcol_v, ibcol_v, i_val, chunk_buf,
             acc_v, uval_v, mean_v, out_t, row_s, sem):
    cid = lax.axis_index("c")
    sid = lax.axis_index("s")

    pltpu.sync_copy(u_idx_hbm.at[sid], u_idx_v)
    pltpu.sync_copy(i_idx_hbm.at[sid], i_idx_v)
    pltpu.sync_copy(i_row_hbm.at[sid], i_row_v)
    pltpu.sync_copy(i_col_hbm.at[sid], i_col_v)
    pltpu.sync_copy(mean_hbm, mean_v)

    def zero(g, carry):
        sl = pl.ds(g * LANES, LANES)
        acc_v[sl] = jnp.zeros((LANES,), jnp.float32)
        u = u_idx_v[sl]
        i = i_idx_v[sl]
        ubrow_v[sl] = lax.shift_right_logical(u, 7)
        ibrow_v[sl] = lax.shift_right_logical(i, 7)
        ubcol_v[sl] = jnp.bitwise_and(u, CHUNK - 1)
        ibcol_v[sl] = jnp.bitwise_and(i, CHUNK - 1)
        return carry
    lax.fori_loop(0, GROUPS, zero, 0)

    dim_base = cid * D_PER_C

    def item_chunk(cc, carry):
        pltpu.async_copy(i_tab.at[i_row_v.at[pl.ds(cc * CHUNK, CHUNK)]],
                         chunk_buf, sem).wait()

        def dloop(d, carry2):
            def gloop(g, carry3):
                row = g * LANES + lax.iota(jnp.int32, LANES)
                col = i_col_v[pl.ds(cc * CHUNK + g * LANES, LANES)] \
                    + dim_base + d
                i_val[pl.ds(d * B_PER_S + cc * CHUNK + g * LANES, LANES)] = (
                    plsc.load_gather(chunk_buf, [row, col]))
                return carry3
            return lax.fori_loop(0, CHUNK // LANES, gloop, carry2)

        return lax.fori_loop(0, D_PER_C, dloop, carry)

    lax.fori_loop(0, N_CHUNKS, item_chunk, 0)

    def dim(c, carry):
        @pl.when(sid == 0)
        def _stage():
            pltpu.sync_copy(uT.at[dim_base + c], row_s)
        plsc.subcore_barrier()

        copies = [pltpu.async_copy(
            row_s.at[u_idx_v.at[pl.ds(cc * CHUNK, CHUNK)]],
            uval_v.at[pl.ds(cc * CHUNK, CHUNK)], sem) for cc in range(N_CHUNKS)]
        for cp in copies:
            cp.wait()

        def gloop(g, carry2):
            sl = pl.ds(g * LANES, LANES)
            acc_v[sl] = acc_v[sl] + (uval_v[sl]
                                     * i_val[pl.ds(c * B_PER_S + g * LANES,
                                                   LANES)])
            return carry2
        lax.fori_loop(0, GROUPS, gloop, 0)
        plsc.subcore_barrier()
        return carry

    lax.fori_loop(0, D_PER_C, dim, 0)

    @pl.when(cid == 0)
    def _biases():
        for tab, row_ref, col_ref in ((ub_tab, ubrow_v, ubcol_v),
                                      (ib_tab, ibrow_v, ibcol_v)):
            def bias_chunk(cc, carry, tab=tab, row_ref=row_ref,
                           col_ref=col_ref):
                pltpu.async_copy(
                    tab.at[row_ref.at[pl.ds(cc * CHUNK, CHUNK)]],
                    chunk_buf, sem).wait()

                def badd(g, carry2):
                    row = g * LANES + lax.iota(jnp.int32, LANES)
                    fsl = pl.ds(cc * CHUNK + g * LANES, LANES)
                    acc_v[fsl] = acc_v[fsl] + plsc.load_gather(
                        chunk_buf, [row, col_ref[fsl]])
                    return carry2
                return lax.fori_loop(0, CHUNK // LANES, badd, carry)

            lax.fori_loop(0, N_CHUNKS, bias_chunk, 0)

        mean_vec = mean_v[...]

        def madd(g, carry):
            sl = pl.ds(g * LANES, LANES)
            acc_v[sl] = acc_v[sl] + mean_vec
            return carry
        lax.fori_loop(0, GROUPS, madd, 0)

    for r in range(N_CHUNKS):
        def wloop(g, carry, r=r):
            out_t[r, pl.ds(g * LANES, LANES)] = (
                acc_v[pl.ds(r * CHUNK + g * LANES, LANES)])
            return carry
        lax.fori_loop(0, CHUNK // LANES, wloop, 0)

    pltpu.sync_copy(
        out_t, out_hbm.at[pl.ds((cid * NUM_SUBCORES + sid) * N_CHUNKS,
                                N_CHUNKS)])


@jax.jit
def _mf_sc(uT, i_tab, ub_tab, ib_tab, u_idx, i_idx, i_row, i_col, mean):
    mesh = plsc.VectorSubcoreMesh(core_axis_name="c", subcore_axis_name="s",
                                  num_cores=NUM_CORES, num_subcores=NUM_SUBCORES)
    run = functools.partial(
        pl.kernel,
        out_type=jax.ShapeDtypeStruct(
            (NUM_CORES * NUM_SUBCORES * N_CHUNKS, CHUNK), jnp.float32),
        mesh=mesh,
        scratch_types=[
            pltpu.VMEM((B_PER_S,), jnp.int32),
            pltpu.VMEM((B_PER_S,), jnp.int32),
            pltpu.VMEM((B_PER_S,), jnp.int32),
            pltpu.VMEM((B_PER_S,), jnp.int32),
            pltpu.VMEM((B_PER_S,), jnp.int32),
            pltpu.VMEM((B_PER_S,), jnp.int32),
            pltpu.VMEM((B_PER_S,), jnp.int32),
            pltpu.VMEM((B_PER_S,), jnp.int32),
            pltpu.VMEM((D_PER_C * B_PER_S,), jnp.float32),
            pltpu.VMEM((CHUNK, CHUNK), jnp.float32),
            pltpu.VMEM((B_PER_S,), jnp.float32),
            pltpu.VMEM((B_PER_S,), jnp.float32),
            pltpu.VMEM((LANES,), jnp.float32),
            pltpu.VMEM((N_CHUNKS, CHUNK), jnp.float32),
            pltpu.VMEM_SHARED((NUM_USERS,), jnp.float32),
            pltpu.SemaphoreType.DMA,
        ],
        compiler_params=pltpu.CompilerParams(needs_layout_passes=False,
                                             use_tc_tiling_on_sc=True),
    )(_mf_body)
    return run(uT, i_tab, ub_tab, ib_tab, u_idx, i_idx, i_row, i_col, mean)


def kernel(u_id, i_id, user_emb, user_bias, item_emb, item_bias, mean):
    u32 = u_id.astype(jnp.int32)
    i32 = i_id.astype(jnp.int32)
    u_idx = u32.reshape(NUM_SUBCORES, B_PER_S)
    i_idx = i32.reshape(NUM_SUBCORES, B_PER_S)
    i_row = (i32 >> 1).reshape(NUM_SUBCORES, B_PER_S)
    i_col = ((i32 & 1) << 6).reshape(NUM_SUBCORES, B_PER_S)
    uT = user_emb.T
    i_tab = item_emb.reshape(-1, CHUNK)
    ub_tab = jnp.pad(user_bias.reshape(-1),
                     (0, NUM_USERS_PAD - NUM_USERS)).reshape(-1, CHUNK)
    ib_tab = jnp.pad(item_bias.reshape(-1),
                     (0, NUM_ITEMS_PAD - NUM_ITEMS)).reshape(-1, CHUNK)
    mean16 = jnp.broadcast_to(mean.astype(jnp.float32).reshape(1), (LANES,))
    out = _mf_sc(uT, i_tab, ub_tab, ib_tab, u_idx, i_idx, i_row, i_col, mean16)
    part = out.reshape(NUM_CORES, BATCH)
    return part[0] + part[1]
